# baseline (speedup 1.0000x reference)
import functools

import jax
import jax.numpy as jnp
from jax import lax
from jax.experimental import pallas as pl
from jax.experimental.pallas import tpu as pltpu
from jax.experimental.pallas import tpu_sc as plsc

N_NODES = 10000
N_PAD = 10240
E_RAW = 320000
E = E_RAW + N_NODES
E_PAD = 331776
CH = 64
NC, NS = 2, 16
NW = NC * NS

_MESH = dict(core_axis_name="c", subcore_axis_name="s", num_cores=NC,
             num_subcores=NS)


def _iota16():
    return lax.iota(jnp.int32, 16)


def _full16(v):
    return jnp.full((16,), v, jnp.int32)


def _tc_body(act, fi2, fo, blk, alo_ref, ahi_ref, blo_ref, bhi_ref,
             wll_ref, wlh_ref, wrl_ref, wrh_ref,
             xll_ref, xlh_ref, xrl_ref, xrh_ref):
    hlo = alo_ref[...]
    hhi = ahi_ref[...]
    if act:
        hlo = jax.nn.relu(hlo + blo_ref[0:1, :])
        hhi = jax.nn.relu(hhi + bhi_ref[0:1, :])
    xl = (jnp.dot(hlo, wll_ref[...], preferred_element_type=jnp.float32)
          + jnp.dot(hhi, wlh_ref[...], preferred_element_type=jnp.float32))
    xr = (jnp.dot(hlo, wrl_ref[...], preferred_element_type=jnp.float32)
          + jnp.dot(hhi, wrh_ref[...], preferred_element_type=jnp.float32))
    fo2 = fo // 2
    xll_ref[...] = xl[:, :fo2]
    xlh_ref[...] = xl[:, fo2:]
    xrl_ref[...] = xr[:, :fo2]
    xrh_ref[...] = xr[:, fo2:]


def _tc_transform(agg_lo, agg_hi, b, Wl, Wr, act):
    fi2 = agg_lo.shape[1]
    fi = 2 * fi2
    fo = Wl.shape[1]
    fo2 = fo // 2
    blk = 512
    grid = (N_PAD // blk,)
    bb = jnp.broadcast_to(b, (8, fi))
    blo, bhi = bb[:, :fi2], bb[:, fi2:]
    full = lambda r, c: pl.BlockSpec((r, c), lambda i: (0, 0))
    outs = pl.pallas_call(
        functools.partial(_tc_body, act, fi2, fo, blk),
        grid=grid,
        in_specs=[
            pl.BlockSpec((blk, fi2), lambda i: (i, 0)),
            pl.BlockSpec((blk, fi2), lambda i: (i, 0)),
            full(8, fi2), full(8, fi2),
            full(fi2, fo), full(fi2, fo), full(fi2, fo), full(fi2, fo),
        ],
        out_specs=[pl.BlockSpec((blk, fo2), lambda i: (i, 0))] * 4,
        out_shape=[jax.ShapeDtypeStruct((N_PAD, fo2), jnp.float32)] * 4,
    )(agg_lo, agg_hi, blo, bhi, Wl[:fi2], Wl[fi2:], Wr[:fi2], Wr[fi2:])
    xll, xlh, xrl, xrh = outs
    return (jnp.concatenate([xll, xlh], axis=0),
            jnp.concatenate([xrl, xrh], axis=0))


def _sc1_body(fo2, nch, ipack_hbm, xlcat, xrcat, att_hbm, z_hbm,
              ex_hbm, dpart_hbm,
              ipack, bsl, bsh, bdl, bdh, attv, ex_all,
              dspm, sem_i, sem_g):
    c = lax.axis_index("c")
    s = lax.axis_index("s")
    wid = s * NC + c
    nps = N_PAD // NS
    row0 = wid * nch

    pltpu.sync_copy(att_hbm, attv)
    pltpu.sync_copy(z_hbm, dspm.at[pl.ds(s * nps, nps)])
    plsc.subcore_barrier()

    bufs = (bsl, bsh, bdl, bdh)
    nchm1 = nch - 1

    def fire_idx(p, i):
        r = row0 + jnp.minimum(i, nchm1)
        pltpu.async_copy(ipack_hbm.at[r], ipack.at[p], sem_i)

    def wait_idx():
        pltpu.make_async_copy(ipack_hbm.at[row0], ipack.at[0], sem_i).wait()

    def fire_gather(p):
        srcs = (xlcat, xlcat, xrcat, xrcat)
        for k in range(4):
            pltpu.async_copy(srcs[k].at[ipack.at[p, k]], bufs[k].at[p],
                             sem_g)

    def wait_gather():
        for k in range(4):
            pltpu.make_async_copy(xlcat.at[ipack.at[0, 0]], bufs[k].at[0],
                                  sem_g).wait()

    def compute(p, i, j):
        del j

        def acc_half(blo, bhi, aoff, accs):
            def fbody(f, a):
                af = plsc.load_gather(attv, [_full16(f + aoff)])
                out = []
                for g in range(CH // 16):
                    ridx = _full16(g * 16) + _iota16()
                    fv = _full16(f)
                    v = (plsc.load_gather(blo.at[p], [ridx, fv])
                         + plsc.load_gather(bhi.at[p], [ridx, fv]))
                    lr = jnp.maximum(v, 0.2 * v)
                    out.append(a[g] + af * lr)
                return tuple(out)
            return plsc.parallel_loop(0, fo2, carry=accs, unroll=8)(fbody)

        z = jnp.zeros((16,), jnp.float32)
        accs = (z,) * (CH // 16)
        accs = acc_half(bsl, bdl, 0, accs)
        accs = acc_half(bsh, bdh, fo2, accs)
        for g in range(CH // 16):
            ex_all[i, pl.ds(g * 16, 16)] = jnp.exp(accs[g])
        pltpu.sync_copy(ex_all.at[i], dspm.at[ipack.at[p, 2]], add=True)

    fire_idx(0, 0)
    wait_idx()
    fire_gather(0)
    fire_idx(1, 1)

    def jbody(j, _):
        for p in range(2):
            i = 2 * j + p
            wait_idx()
            fire_gather(1 - p)
            wait_gather()
            compute(p, i, j)
            fire_idx(p, i + 2)
        return 0

    lax.fori_loop(0, nch // 2, jbody, 0)
    wait_idx()
    wait_gather()
    pltpu.sync_copy(ex_all, ex_hbm.at[pl.ds(row0, nch), :])
    plsc.subcore_barrier()
    pltpu.sync_copy(dspm.at[pl.ds(s * nps, nps)],
                    dpart_hbm.at[pl.ds(c * N_PAD + s * nps, nps)])


@functools.lru_cache(maxsize=None)
def _mk_sc1(fo2):
    nch = E_PAD // NW // CH
    return pl.kernel(
        functools.partial(_sc1_body, fo2, nch),
        out_type=[jax.ShapeDtypeStruct((E_PAD // CH, CH), jnp.float32),
                  jax.ShapeDtypeStruct((NC * N_PAD,), jnp.float32)],
        mesh=plsc.VectorSubcoreMesh(**_MESH),
        compiler_params=pltpu.CompilerParams(needs_layout_passes=False,
                                             use_tc_tiling_on_sc=False),
        scratch_types=[
            pltpu.VMEM((2, 4, CH), jnp.int32),
            pltpu.VMEM((2, CH, fo2), jnp.float32),
            pltpu.VMEM((2, CH, fo2), jnp.float32),
            pltpu.VMEM((2, CH, fo2), jnp.float32),
            pltpu.VMEM((2, CH, fo2), jnp.float32),
            pltpu.VMEM((2 * fo2,), jnp.float32),
            pltpu.VMEM((E_PAD // NW // CH, CH), jnp.float32),
            pltpu.VMEM_SHARED((N_PAD,), jnp.float32),
            pltpu.SemaphoreType.DMA,
            pltpu.SemaphoreType.DMA,
        ],
    )


def _sc1(ipack, xlcat, xrcat, att):
    z = jnp.zeros((N_PAD // NS,), jnp.float32)
    return _mk_sc1(xlcat.shape[1])(ipack, xlcat, xrcat, att, z)


def _sc2_body(fo2, nch, soff_hbm, dst_hbm, ex_hbm, dpart_hbm, xlcat,
              zblk_hbm, out_hbm,
              sidx, didx, exb, alph, rows, denom, tmpd, acc,
              sem_i, sem_g):
    c = lax.axis_index("c")
    s = lax.axis_index("s")
    nps = N_PAD // NS
    base0 = s * nch * CH
    soff0 = c * E_PAD + base0

    pltpu.sync_copy(zblk_hbm, acc.at[pl.ds(s * nps, nps)])
    pltpu.sync_copy(dpart_hbm.at[pl.ds(0, N_PAD)], denom)
    pltpu.sync_copy(dpart_hbm.at[pl.ds(N_PAD, N_PAD)], tmpd)

    def dmerge(i):
        sl = pl.ds(i * 16, 16)
        denom[sl] = denom[sl] + tmpd[sl]
    plsc.parallel_loop(0, N_PAD // 16, unroll=8)(dmerge)
    plsc.subcore_barrier()

    maxo = (nch - 1) * CH

    def fire_idx(p, i):
        o = jnp.minimum(i * CH, maxo)
        pltpu.async_copy(soff_hbm.at[pl.ds(soff0 + o, CH)], sidx.at[p],
                         sem_i)
        pltpu.async_copy(dst_hbm.at[pl.ds(base0 + o, CH)], didx.at[p],
                         sem_i)
        pltpu.async_copy(ex_hbm.at[pl.ds(base0 + o, CH)], exb.at[p], sem_i)

    def wait_idx():
        pltpu.make_async_copy(soff_hbm.at[pl.ds(soff0, CH)], sidx.at[0],
                              sem_i).wait()
        pltpu.make_async_copy(dst_hbm.at[pl.ds(base0, CH)], didx.at[0],
                              sem_i).wait()
        pltpu.make_async_copy(ex_hbm.at[pl.ds(base0, CH)], exb.at[0],
                              sem_i).wait()

    def fire_gather(p):
        pltpu.async_copy(xlcat.at[sidx.at[p]], rows.at[p], sem_g)

    def wait_gather():
        pltpu.make_async_copy(xlcat.at[sidx.at[0]], rows.at[0],
                              sem_g).wait()

    def compute(p, i):
        for g in range(CH // 16):
            sl = pl.ds(g * 16, 16)
            den = plsc.load_gather(denom, [didx[p, sl]])
            alph[p, sl] = exb[p, sl] / den

        def fscale(f):
            fv = _full16(f)
            for g in range(CH // 16):
                ridx = _full16(g * 16) + _iota16()
                v = (plsc.load_gather(rows.at[p], [ridx, fv])
                     * alph[p, pl.ds(g * 16, 16)])
                plsc.store_scatter(rows.at[p], [ridx, fv], v)
        plsc.parallel_loop(0, fo2, unroll=4)(fscale)
        pltpu.sync_copy(rows.at[p], acc.at[didx.at[p]], add=True)

    fire_idx(0, 0)
    wait_idx()
    fire_gather(0)
    fire_idx(1, 1)

    def jbody(j, _):
        for p in range(2):
            i = 2 * j + p
            wait_idx()
            fire_gather(1 - p)
            wait_gather()
            compute(p, i)
            fire_idx(p, i + 2)
        return 0

    lax.fori_loop(0, nch // 2, jbody, 0)
    wait_idx()
    wait_gather()
    plsc.subcore_barrier()
    pltpu.sync_copy(acc.at[pl.ds(s * nps, nps)],
                    out_hbm.at[pl.ds(c * N_PAD + s * nps, nps)])


@functools.lru_cache(maxsize=None)
def _mk_sc2(fo2):
    nch = E_PAD // NS // CH
    return pl.kernel(
        functools.partial(_sc2_body, fo2, nch),
        out_type=jax.ShapeDtypeStruct((NC * N_PAD, fo2), jnp.float32),
        mesh=plsc.VectorSubcoreMesh(**_MESH),
        compiler_params=pltpu.CompilerParams(needs_layout_passes=False,
                                             use_tc_tiling_on_sc=False),
        scratch_types=[
            pltpu.VMEM((2, CH), jnp.int32),
            pltpu.VMEM((2, CH), jnp.int32),
            pltpu.VMEM((2, CH), jnp.float32),
            pltpu.VMEM((2, CH), jnp.float32),
            pltpu.VMEM((2, CH, fo2), jnp.float32),
            pltpu.VMEM((N_PAD,), jnp.float32),
            pltpu.VMEM((N_PAD,), jnp.float32),
            pltpu.VMEM_SHARED((N_PAD, fo2), jnp.float32),
            pltpu.SemaphoreType.DMA,
            pltpu.SemaphoreType.DMA,
        ],
    )


def _sc2(soff, dst, ex, dpart, xlcat):
    fo2 = xlcat.shape[1]
    zblk = jnp.zeros((N_PAD // NS, fo2), jnp.float32)
    return _mk_sc2(fo2)(soff, dst, ex, dpart, xlcat, zblk)


def _root_body(fo2, root_hbm, agg_hbm, b_hbm, out_lo_hbm, out_hi_hbm,
               ridx, ridx2, rows_lo, rows_hi, blo, bhi, sem):
    c = lax.axis_index("c")
    s = lax.axis_index("s")
    wid = s * NC + c
    nper = 1024 // NW
    base = wid * nper
    pltpu.sync_copy(root_hbm.at[pl.ds(base, nper)], ridx)
    for g in range(nper // 16):
        sl = pl.ds(g * 16, 16)
        ridx2[sl] = ridx[sl] + N_PAD
    d1 = pltpu.async_copy(agg_hbm.at[ridx], rows_lo, sem)
    d2 = pltpu.async_copy(agg_hbm.at[ridx2], rows_hi, sem)
    pltpu.sync_copy(b_hbm.at[pl.ds(0, fo2)], blo)
    pltpu.sync_copy(b_hbm.at[pl.ds(fo2, fo2)], bhi)
    d1.wait(); d2.wait()

    def fbias(f, _):
        fv = _full16(f)
        bl = plsc.load_gather(blo, [fv])
        bh = plsc.load_gather(bhi, [fv])
        for g in range(nper // 16):
            ridxg = _full16(g * 16) + _iota16()
            plsc.store_scatter(rows_lo, [ridxg, fv],
                               plsc.load_gather(rows_lo, [ridxg, fv]) + bl)
            plsc.store_scatter(rows_hi, [ridxg, fv],
                               plsc.load_gather(rows_hi, [ridxg, fv]) + bh)
        return 0
    lax.fori_loop(0, fo2, fbias, 0)
    pltpu.sync_copy(rows_lo, out_lo_hbm.at[pl.ds(base, nper)])
    pltpu.sync_copy(rows_hi, out_hi_hbm.at[pl.ds(base, nper)])


def _root_gather(root, agg, b):
    fo2 = agg.shape[1]
    nper = 1024 // NW
    f = pl.kernel(
        functools.partial(_root_body, fo2),
        out_type=[jax.ShapeDtypeStruct((1024, fo2), jnp.float32),
                  jax.ShapeDtypeStruct((1024, fo2), jnp.float32)],
        mesh=plsc.VectorSubcoreMesh(**_MESH),
        compiler_params=pltpu.CompilerParams(needs_layout_passes=False, use_tc_tiling_on_sc=False),
        scratch_types=[
            pltpu.VMEM((nper,), jnp.int32), pltpu.VMEM((nper,), jnp.int32),
            pltpu.VMEM((nper, fo2), jnp.float32),
            pltpu.VMEM((nper, fo2), jnp.float32),
            pltpu.VMEM((fo2,), jnp.float32), pltpu.VMEM((fo2,), jnp.float32),
            pltpu.SemaphoreType.DMA,
        ],
    )
    out_lo, out_hi = f(root, agg, b)
    return jnp.concatenate([out_lo, out_hi], axis=1)


def _layer(agg_lo, agg_hi, ipack, soff, dst, Wl, Wr, att, b, act):
    xlcat, xrcat = _tc_transform(agg_lo, agg_hi, b, Wl, Wr, act)
    ex, dpart = _sc1(ipack, xlcat, xrcat, att)
    agg = _sc2(soff, dst, ex.reshape(-1), dpart, xlcat)
    return agg[:N_PAD], agg[N_PAD:], agg


def kernel(x, edge_index, root_index, Wl0, Wr0, a0, b0, Wl1, Wr1, a1, b1,
           Wl2, Wr2, a2, b2, Wl3, Wr3, a3, b3, Wl4, Wr4, a4, b4):
    n = x.shape[0]
    loop = jnp.arange(n, dtype=jnp.int32)
    src = jnp.concatenate([edge_index[0], loop])
    dst = jnp.concatenate([edge_index[1], loop])
    pad = jnp.full((E_PAD - E,), N_PAD - 1, jnp.int32)
    src = jnp.concatenate([src, pad])
    dst = jnp.concatenate([dst, pad])
    ipack = jnp.stack([src.reshape(-1, CH), (src + N_PAD).reshape(-1, CH),
                       dst.reshape(-1, CH), (dst + N_PAD).reshape(-1, CH)],
                      axis=1)
    soff = jnp.concatenate([src, src + N_PAD])

    xp = jnp.pad(x, ((0, N_PAD - n), (0, 0)))
    alo, ahi = xp[:, :64], xp[:, 64:]

    alo, ahi, _ = _layer(alo, ahi, ipack, soff, dst, Wl0, Wr0, a0,
                         jnp.zeros((128,), jnp.float32), act=False)
    alo, ahi, _ = _layer(alo, ahi, ipack, soff, dst, Wl1, Wr1, a1, b0, act=True)
    alo, ahi, _ = _layer(alo, ahi, ipack, soff, dst, Wl2, Wr2, a2, b1, act=True)
    alo, ahi, _ = _layer(alo, ahi, ipack, soff, dst, Wl3, Wr3, a3, b2, act=True)
    _, _, agg4 = _layer(alo, ahi, ipack, soff, dst, Wl4, Wr4, a4, b3, act=True)

    return _root_gather(root_index, agg4, b4)


# speedup vs baseline: 1.0656x; 1.0656x over previous
import functools

import jax
import jax.numpy as jnp
from jax import lax
from jax.experimental import pallas as pl
from jax.experimental.pallas import tpu as pltpu
from jax.experimental.pallas import tpu_sc as plsc

N_NODES = 10000
N_PAD = 10240
E_RAW = 320000
E = E_RAW + N_NODES
E_PAD = 331776
CH = 64
NC, NS = 2, 16
NW = NC * NS

_MESH = dict(core_axis_name="c", subcore_axis_name="s", num_cores=NC,
             num_subcores=NS)


def _iota16():
    return lax.iota(jnp.int32, 16)


def _full16(v):
    return jnp.full((16,), v, jnp.int32)


def _tc_body(act, fi2, fo, blk, alo_ref, ahi_ref, blo_ref, bhi_ref,
             wll_ref, wlh_ref, wrl_ref, wrh_ref,
             xll_ref, xlh_ref, xrl_ref, xrh_ref):
    hlo = alo_ref[...]
    hhi = ahi_ref[...]
    if act:
        hlo = jax.nn.relu(hlo + blo_ref[0:1, :])
        hhi = jax.nn.relu(hhi + bhi_ref[0:1, :])
    xl = (jnp.dot(hlo, wll_ref[...], preferred_element_type=jnp.float32)
          + jnp.dot(hhi, wlh_ref[...], preferred_element_type=jnp.float32))
    xr = (jnp.dot(hlo, wrl_ref[...], preferred_element_type=jnp.float32)
          + jnp.dot(hhi, wrh_ref[...], preferred_element_type=jnp.float32))
    fo2 = fo // 2
    xll_ref[...] = xl[:, :fo2]
    xlh_ref[...] = xl[:, fo2:]
    xrl_ref[...] = xr[:, :fo2]
    xrh_ref[...] = xr[:, fo2:]


def _tc_transform(agg_lo, agg_hi, b, Wl, Wr, act):
    fi2 = agg_lo.shape[1]
    fi = 2 * fi2
    fo = Wl.shape[1]
    fo2 = fo // 2
    blk = 512
    grid = (N_PAD // blk,)
    bb = jnp.broadcast_to(b, (8, fi))
    blo, bhi = bb[:, :fi2], bb[:, fi2:]
    full = lambda r, c: pl.BlockSpec((r, c), lambda i: (0, 0))
    outs = pl.pallas_call(
        functools.partial(_tc_body, act, fi2, fo, blk),
        grid=grid,
        in_specs=[
            pl.BlockSpec((blk, fi2), lambda i: (i, 0)),
            pl.BlockSpec((blk, fi2), lambda i: (i, 0)),
            full(8, fi2), full(8, fi2),
            full(fi2, fo), full(fi2, fo), full(fi2, fo), full(fi2, fo),
        ],
        out_specs=[pl.BlockSpec((blk, fo2), lambda i: (i, 0))] * 4,
        out_shape=[jax.ShapeDtypeStruct((N_PAD, fo2), jnp.float32)] * 4,
    )(agg_lo, agg_hi, blo, bhi, Wl[:fi2], Wl[fi2:], Wr[:fi2], Wr[fi2:])
    xll, xlh, xrl, xrh = outs
    return (jnp.concatenate([xll, xlh], axis=0),
            jnp.concatenate([xrl, xrh], axis=0))


def _sc1_body(fo2, nch, ipack_hbm, xlcat, xrcat, att_hbm, z_hbm,
              ex_hbm, dpart_hbm,
              ipack, bsl, bsh, bdl, bdh, attv, ex_all,
              dspm, sem_i, sem_g):
    c = lax.axis_index("c")
    s = lax.axis_index("s")
    wid = s * NC + c
    nps = N_PAD // NS
    row0 = wid * nch

    pltpu.sync_copy(att_hbm, attv)
    pltpu.sync_copy(z_hbm, dspm.at[pl.ds(s * nps, nps)])
    plsc.subcore_barrier()

    bufs = (bsl, bsh, bdl, bdh)
    nchm1 = nch - 1

    def fire_idx(p, i):
        r = row0 + jnp.minimum(i, nchm1)
        pltpu.async_copy(ipack_hbm.at[r], ipack.at[p], sem_i)

    def wait_idx():
        pltpu.make_async_copy(ipack_hbm.at[row0], ipack.at[0], sem_i).wait()

    def fire_gather(p):
        srcs = (xlcat, xlcat, xrcat, xrcat)
        for k in range(4):
            pltpu.async_copy(srcs[k].at[ipack.at[p, k]], bufs[k].at[p],
                             sem_g)

    def wait_gather():
        for k in range(4):
            pltpu.make_async_copy(xlcat.at[ipack.at[0, 0]], bufs[k].at[0],
                                  sem_g).wait()

    def compute(p, i, j):
        del j

        def acc_half(blo, bhi, aoff, accs):
            def fbody(f, a):
                af = plsc.load_gather(attv, [_full16(f + aoff)])
                out = []
                for g in range(CH // 16):
                    ridx = _full16(g * 16) + _iota16()
                    fv = _full16(f)
                    v = (plsc.load_gather(blo.at[p], [ridx, fv])
                         + plsc.load_gather(bhi.at[p], [ridx, fv]))
                    lr = jnp.maximum(v, 0.2 * v)
                    out.append(a[g] + af * lr)
                return tuple(out)
            return plsc.parallel_loop(0, fo2, carry=accs, unroll=4)(fbody)

        z = jnp.zeros((16,), jnp.float32)
        accs = (z,) * (CH // 16)
        accs = acc_half(bsl, bdl, 0, accs)
        accs = acc_half(bsh, bdh, fo2, accs)
        for g in range(CH // 16):
            ex_all[i, pl.ds(g * 16, 16)] = jnp.exp(accs[g])
        pltpu.sync_copy(ex_all.at[i], dspm.at[ipack.at[p, 2]], add=True)

    fire_idx(0, 0)
    wait_idx()
    fire_gather(0)
    fire_idx(1, 1)

    def jbody(j, _):
        for p in range(2):
            i = 2 * j + p
            wait_idx()
            fire_gather(1 - p)
            wait_gather()
            compute(p, i, j)
            fire_idx(p, i + 2)
        return 0

    lax.fori_loop(0, nch // 2, jbody, 0)
    wait_idx()
    wait_gather()
    pltpu.sync_copy(ex_all, ex_hbm.at[pl.ds(row0, nch), :])
    plsc.subcore_barrier()
    pltpu.sync_copy(dspm.at[pl.ds(s * nps, nps)],
                    dpart_hbm.at[pl.ds(c * N_PAD + s * nps, nps)])


@functools.lru_cache(maxsize=None)
def _mk_sc1(fo2):
    nch = E_PAD // NW // CH
    return pl.kernel(
        functools.partial(_sc1_body, fo2, nch),
        out_type=[jax.ShapeDtypeStruct((E_PAD // CH, CH), jnp.float32),
                  jax.ShapeDtypeStruct((NC * N_PAD,), jnp.float32)],
        mesh=plsc.VectorSubcoreMesh(**_MESH),
        compiler_params=pltpu.CompilerParams(needs_layout_passes=False,
                                             use_tc_tiling_on_sc=False),
        scratch_types=[
            pltpu.VMEM((2, 4, CH), jnp.int32),
            pltpu.VMEM((2, CH, fo2), jnp.float32),
            pltpu.VMEM((2, CH, fo2), jnp.float32),
            pltpu.VMEM((2, CH, fo2), jnp.float32),
            pltpu.VMEM((2, CH, fo2), jnp.float32),
            pltpu.VMEM((2 * fo2,), jnp.float32),
            pltpu.VMEM((E_PAD // NW // CH, CH), jnp.float32),
            pltpu.VMEM_SHARED((N_PAD,), jnp.float32),
            pltpu.SemaphoreType.DMA,
            pltpu.SemaphoreType.DMA,
        ],
    )


def _sc1(ipack, xlcat, xrcat, att):
    z = jnp.zeros((N_PAD // NS,), jnp.float32)
    return _mk_sc1(xlcat.shape[1])(ipack, xlcat, xrcat, att, z)


def _sc2_body(fo2, nch, soff_hbm, dst_hbm, ex_hbm, dpart_hbm, xlcat,
              zblk_hbm, out_hbm,
              sidx, didx, exb, alph, rows, denom, tmpd, acc,
              sem_i, sem_g):
    c = lax.axis_index("c")
    s = lax.axis_index("s")
    nps = N_PAD // NS
    base0 = s * nch * CH
    soff0 = c * E_PAD + base0

    pltpu.sync_copy(zblk_hbm, acc.at[pl.ds(s * nps, nps)])
    pltpu.sync_copy(dpart_hbm.at[pl.ds(0, N_PAD)], denom)
    pltpu.sync_copy(dpart_hbm.at[pl.ds(N_PAD, N_PAD)], tmpd)

    def dmerge(i):
        sl = pl.ds(i * 16, 16)
        denom[sl] = denom[sl] + tmpd[sl]
    plsc.parallel_loop(0, N_PAD // 16, unroll=8)(dmerge)
    plsc.subcore_barrier()

    maxo = (nch - 1) * CH

    def fire_idx(p, i):
        o = jnp.minimum(i * CH, maxo)
        pltpu.async_copy(soff_hbm.at[pl.ds(soff0 + o, CH)], sidx.at[p],
                         sem_i)
        pltpu.async_copy(dst_hbm.at[pl.ds(base0 + o, CH)], didx.at[p],
                         sem_i)
        pltpu.async_copy(ex_hbm.at[pl.ds(base0 + o, CH)], exb.at[p], sem_i)

    def wait_idx():
        pltpu.make_async_copy(soff_hbm.at[pl.ds(soff0, CH)], sidx.at[0],
                              sem_i).wait()
        pltpu.make_async_copy(dst_hbm.at[pl.ds(base0, CH)], didx.at[0],
                              sem_i).wait()
        pltpu.make_async_copy(ex_hbm.at[pl.ds(base0, CH)], exb.at[0],
                              sem_i).wait()

    def fire_gather(p):
        pltpu.async_copy(xlcat.at[sidx.at[p]], rows.at[p], sem_g)

    def wait_gather():
        pltpu.make_async_copy(xlcat.at[sidx.at[0]], rows.at[0],
                              sem_g).wait()

    def compute(p, i):
        for g in range(CH // 16):
            sl = pl.ds(g * 16, 16)
            den = plsc.load_gather(denom, [didx[p, sl]])
            alph[p, sl] = exb[p, sl] / den

        def fscale(f):
            fv = _full16(f)
            for g in range(CH // 16):
                ridx = _full16(g * 16) + _iota16()
                v = (plsc.load_gather(rows.at[p], [ridx, fv])
                     * alph[p, pl.ds(g * 16, 16)])
                plsc.store_scatter(rows.at[p], [ridx, fv], v)
        plsc.parallel_loop(0, fo2, unroll=4)(fscale)
        pltpu.sync_copy(rows.at[p], acc.at[didx.at[p]], add=True)

    fire_idx(0, 0)
    wait_idx()
    fire_gather(0)
    fire_idx(1, 1)

    def jbody(j, _):
        for p in range(2):
            i = 2 * j + p
            wait_idx()
            fire_gather(1 - p)
            wait_gather()
            compute(p, i)
            fire_idx(p, i + 2)
        return 0

    lax.fori_loop(0, nch // 2, jbody, 0)
    wait_idx()
    wait_gather()
    plsc.subcore_barrier()
    pltpu.sync_copy(acc.at[pl.ds(s * nps, nps)],
                    out_hbm.at[pl.ds(c * N_PAD + s * nps, nps)])


@functools.lru_cache(maxsize=None)
def _mk_sc2(fo2):
    nch = E_PAD // NS // CH
    return pl.kernel(
        functools.partial(_sc2_body, fo2, nch),
        out_type=jax.ShapeDtypeStruct((NC * N_PAD, fo2), jnp.float32),
        mesh=plsc.VectorSubcoreMesh(**_MESH),
        compiler_params=pltpu.CompilerParams(needs_layout_passes=False,
                                             use_tc_tiling_on_sc=False),
        scratch_types=[
            pltpu.VMEM((2, CH), jnp.int32),
            pltpu.VMEM((2, CH), jnp.int32),
            pltpu.VMEM((2, CH), jnp.float32),
            pltpu.VMEM((2, CH), jnp.float32),
            pltpu.VMEM((2, CH, fo2), jnp.float32),
            pltpu.VMEM((N_PAD,), jnp.float32),
            pltpu.VMEM((N_PAD,), jnp.float32),
            pltpu.VMEM_SHARED((N_PAD, fo2), jnp.float32),
            pltpu.SemaphoreType.DMA,
            pltpu.SemaphoreType.DMA,
        ],
    )


def _sc2(soff, dst, ex, dpart, xlcat):
    fo2 = xlcat.shape[1]
    zblk = jnp.zeros((N_PAD // NS, fo2), jnp.float32)
    return _mk_sc2(fo2)(soff, dst, ex, dpart, xlcat, zblk)


def _root_body(fo2, root_hbm, agg_hbm, b_hbm, out_lo_hbm, out_hi_hbm,
               ridx, ridx2, rows_lo, rows_hi, blo, bhi, sem):
    c = lax.axis_index("c")
    s = lax.axis_index("s")
    wid = s * NC + c
    nper = 1024 // NW
    base = wid * nper
    pltpu.sync_copy(root_hbm.at[pl.ds(base, nper)], ridx)
    for g in range(nper // 16):
        sl = pl.ds(g * 16, 16)
        ridx2[sl] = ridx[sl] + N_PAD
    d1 = pltpu.async_copy(agg_hbm.at[ridx], rows_lo, sem)
    d2 = pltpu.async_copy(agg_hbm.at[ridx2], rows_hi, sem)
    pltpu.sync_copy(b_hbm.at[pl.ds(0, fo2)], blo)
    pltpu.sync_copy(b_hbm.at[pl.ds(fo2, fo2)], bhi)
    d1.wait(); d2.wait()

    def fbias(f, _):
        fv = _full16(f)
        bl = plsc.load_gather(blo, [fv])
        bh = plsc.load_gather(bhi, [fv])
        for g in range(nper // 16):
            ridxg = _full16(g * 16) + _iota16()
            plsc.store_scatter(rows_lo, [ridxg, fv],
                               plsc.load_gather(rows_lo, [ridxg, fv]) + bl)
            plsc.store_scatter(rows_hi, [ridxg, fv],
                               plsc.load_gather(rows_hi, [ridxg, fv]) + bh)
        return 0
    lax.fori_loop(0, fo2, fbias, 0)
    pltpu.sync_copy(rows_lo, out_lo_hbm.at[pl.ds(base, nper)])
    pltpu.sync_copy(rows_hi, out_hi_hbm.at[pl.ds(base, nper)])


def _root_gather(root, agg, b):
    fo2 = agg.shape[1]
    nper = 1024 // NW
    f = pl.kernel(
        functools.partial(_root_body, fo2),
        out_type=[jax.ShapeDtypeStruct((1024, fo2), jnp.float32),
                  jax.ShapeDtypeStruct((1024, fo2), jnp.float32)],
        mesh=plsc.VectorSubcoreMesh(**_MESH),
        compiler_params=pltpu.CompilerParams(needs_layout_passes=False, use_tc_tiling_on_sc=False),
        scratch_types=[
            pltpu.VMEM((nper,), jnp.int32), pltpu.VMEM((nper,), jnp.int32),
            pltpu.VMEM((nper, fo2), jnp.float32),
            pltpu.VMEM((nper, fo2), jnp.float32),
            pltpu.VMEM((fo2,), jnp.float32), pltpu.VMEM((fo2,), jnp.float32),
            pltpu.SemaphoreType.DMA,
        ],
    )
    out_lo, out_hi = f(root, agg, b)
    return jnp.concatenate([out_lo, out_hi], axis=1)


def _layer(agg_lo, agg_hi, ipack, soff, dst, Wl, Wr, att, b, act):
    xlcat, xrcat = _tc_transform(agg_lo, agg_hi, b, Wl, Wr, act)
    ex, dpart = _sc1(ipack, xlcat, xrcat, att)
    agg = _sc2(soff, dst, ex.reshape(-1), dpart, xlcat)
    return agg[:N_PAD], agg[N_PAD:], agg


def kernel(x, edge_index, root_index, Wl0, Wr0, a0, b0, Wl1, Wr1, a1, b1,
           Wl2, Wr2, a2, b2, Wl3, Wr3, a3, b3, Wl4, Wr4, a4, b4):
    n = x.shape[0]
    loop = jnp.arange(n, dtype=jnp.int32)
    src = jnp.concatenate([edge_index[0], loop])
    dst = jnp.concatenate([edge_index[1], loop])
    pad = jnp.full((E_PAD - E,), N_PAD - 1, jnp.int32)
    src = jnp.concatenate([src, pad])
    dst = jnp.concatenate([dst, pad])
    ipack = jnp.stack([src.reshape(-1, CH), (src + N_PAD).reshape(-1, CH),
                       dst.reshape(-1, CH), (dst + N_PAD).reshape(-1, CH)],
                      axis=1)
    soff = jnp.concatenate([src, src + N_PAD])

    xp = jnp.pad(x, ((0, N_PAD - n), (0, 0)))
    alo, ahi = xp[:, :64], xp[:, 64:]

    alo, ahi, _ = _layer(alo, ahi, ipack, soff, dst, Wl0, Wr0, a0,
                         jnp.zeros((128,), jnp.float32), act=False)
    alo, ahi, _ = _layer(alo, ahi, ipack, soff, dst, Wl1, Wr1, a1, b0, act=True)
    alo, ahi, _ = _layer(alo, ahi, ipack, soff, dst, Wl2, Wr2, a2, b1, act=True)
    alo, ahi, _ = _layer(alo, ahi, ipack, soff, dst, Wl3, Wr3, a3, b2, act=True)
    _, _, agg4 = _layer(alo, ahi, ipack, soff, dst, Wl4, Wr4, a4, b3, act=True)

    return _root_gather(root_index, agg4, b4)
